# baseline (speedup 1.0000x reference)
import functools

import jax
import jax.numpy as jnp
from jax import lax
from jax.experimental import pallas as pl
from jax.experimental.pallas import tpu as pltpu
from jax.experimental.pallas import tpu_sc as plsc

B = 16384
D = 16
G = 128 // D


def _pack_cols(lo, hi):
    lo_b = lax.bitcast_convert_type(lo.astype(jnp.bfloat16), jnp.uint16)
    hi_b = lax.bitcast_convert_type(hi.astype(jnp.bfloat16), jnp.uint16)
    packed = jnp.bitwise_or(
        lax.shift_left(hi_b.astype(jnp.uint32), jnp.uint32(16)),
        lo_b.astype(jnp.uint32))
    return lax.bitcast_convert_type(packed, jnp.float32)


def _regroup_body(xt_ref, out_ref):
    x = xt_ref[...]
    nq = x.shape[1] // 1024
    ident = jnp.where(
        lax.broadcasted_iota(jnp.int32, (128, 128), 0)
        == lax.broadcasted_iota(jnp.int32, (128, 128), 1), 1.0, 0.0)
    for m in range(nq // 2):
        st = []
        for q in (2 * m, 2 * m + 1):
            s = jnp.concatenate(
                [x[:, q * 1024 + t * 128:q * 1024 + (t + 1) * 128]
                 for t in range(8)], axis=0)
            st.append(lax.dot_general(
                ident, s, (((1,), (1,)), ((), ())),
                preferred_element_type=jnp.float32))
        s2 = jnp.concatenate(st, axis=1)
        packed = [
            _pack_cols(s2[:, t2 * D:t2 * D + 8], s2[:, t2 * D + 8:(t2 + 1) * D])
            for t2 in range(16)]
        out_ref[m * 128:(m + 1) * 128, :] = jnp.concatenate(packed, axis=1)


def _regroup(table_t, blk_i=131072):
    d, v = table_t.shape
    grid = (pl.cdiv(v, blk_i),)
    return pl.pallas_call(
        _regroup_body,
        grid=grid,
        in_specs=[pl.BlockSpec((d, blk_i), lambda i: (0, i))],
        out_specs=pl.BlockSpec((blk_i // 16, 128), lambda i: (i, 0)),
        out_shape=jax.ShapeDtypeStruct(
            (pl.cdiv(v, blk_i) * (blk_i // 16), 128), jnp.float32),
    )(table_t)


def _make_sc_gather(hmode):
    info = plsc.get_sparse_core_info()
    nc, ns = info.num_cores, info.num_subcores
    nw = nc * ns
    bpw = B // nw

    mesh = plsc.VectorSubcoreMesh(core_axis_name="c", subcore_axis_name="s")

    @functools.partial(
        pl.kernel,
        mesh=mesh,
        out_type=jax.ShapeDtypeStruct((B, 128), jnp.float32),
        scratch_types=[
            pltpu.VMEM((bpw,), jnp.int32),
            pltpu.VMEM((bpw,), jnp.int32),
            pltpu.VMEM((128, 128), jnp.float32),
            pltpu.VMEM((128, 128), jnp.float32),
            pltpu.SemaphoreType.DMA,
            pltpu.SemaphoreType.DMA,
        ],
    )
    def sc_gather(
        idx_hbm, tab_hbm, out,
        idx_v, hi_v, rows_a, rows_b, sem_a, sem_b,
    ):
        wid = lax.axis_index("s") * nc + lax.axis_index("c")
        base = wid * bpw
        sl = pl.ds(base, bpw)

        pltpu.sync_copy(idx_hbm.at[sl], idx_v)

        for k in range(bpw // 16):
            ks = pl.ds(k * 16, 16)
            ix = idx_v[ks]
            if hmode:
                hi_v[ks] = jnp.bitwise_or(
                    lax.shift_left(lax.shift_right_logical(ix, 11), 7),
                    jnp.bitwise_and(ix, 127))
            else:
                hi_v[ks] = lax.shift_right_logical(ix, 3)

        nch = bpw // 128
        rows = [rows_a, rows_b]
        sems = [sem_a, sem_b]
        cps = [None, None]
        for t in range(nch):
            cps[t % 2] = pltpu.async_copy(
                tab_hbm.at[hi_v.at[pl.ds(t * 128, 128)]],
                rows[t % 2], sems[t % 2])
            if t > 0:
                cps[(t - 1) % 2].wait()
                pltpu.sync_copy(rows[(t - 1) % 2],
                                out.at[pl.ds(base + (t - 1) * 128, 128)])
        cps[(nch - 1) % 2].wait()
        pltpu.sync_copy(rows[(nch - 1) % 2],
                        out.at[pl.ds(base + (nch - 1) * 128, 128)])

    return sc_gather


_sc_gather = _make_sc_gather(True)
_sc_gather_grouped = _make_sc_gather(False)


def _extract(rows128, idx, hmode=True):
    del hmode
    off = jnp.bitwise_and(lax.shift_right_logical(idx, 7), 15)
    sel = jnp.zeros((rows128.shape[0], 8), jnp.float32)
    for j in range(16):
        sel = sel + jnp.where(off == j, rows128[:, j * 8:(j + 1) * 8], 0.0)
    bits = lax.bitcast_convert_type(sel, jnp.uint32)
    lo = lax.bitcast_convert_type(
        bits.astype(jnp.uint16), jnp.bfloat16).astype(jnp.float32)
    hi = lax.bitcast_convert_type(
        lax.shift_right_logical(bits, jnp.uint32(16)).astype(jnp.uint16),
        jnp.bfloat16).astype(jnp.float32)
    return jnp.concatenate([lo, hi], axis=1)


def _tc_body(u_ref, i_ref, d_ref, eu_ref, ei_ref, ed_ref, year_ref, goh_ref,
             gt_ref, fcw_ref, bias_ref, w1_ref, b1_ref, w2_ref, b2_ref,
             w3_ref, b3_ref, y_ref):
    eu = _extract(eu_ref[...], u_ref[...])
    ei = _extract(ei_ref[...], i_ref[...])
    ed = _extract(ed_ref[...], d_ref[...])
    g = jnp.dot(goh_ref[...], gt_ref[...],
                preferred_element_type=jnp.float32)
    yr = year_ref[...]

    v = eu + ei + ed + g
    s = jnp.sum(v, axis=1, keepdims=True)
    ssq = jnp.sum(v * v, axis=1, keepdims=True)
    second = 0.5 * (s * s - ssq)

    fcw = fcw_ref[...]
    fm = (
        jnp.dot(eu, fcw[0:D], preferred_element_type=jnp.float32)
        + jnp.dot(ei, fcw[D:2 * D], preferred_element_type=jnp.float32)
        + jnp.dot(ed, fcw[3 * D:4 * D], preferred_element_type=jnp.float32)
        + jnp.dot(g, fcw[2 * D:3 * D], preferred_element_type=jnp.float32)
        + bias_ref[...]
        + second
        + yr
    )

    w1 = w1_ref[...]
    p = (
        jnp.dot(eu, w1[0:D], preferred_element_type=jnp.float32)
        + jnp.dot(ei, w1[D:2 * D], preferred_element_type=jnp.float32)
        + jnp.dot(ed, w1[2 * D:3 * D], preferred_element_type=jnp.float32)
        + jnp.dot(g, w1[3 * D:4 * D], preferred_element_type=jnp.float32)
        + yr * w1[4 * D:4 * D + 1]
        + b1_ref[...]
    )
    h1 = jnp.maximum(p, 0.0)
    h2 = jnp.maximum(
        jnp.dot(h1, w2_ref[...], preferred_element_type=jnp.float32)
        + b2_ref[...], 0.0)
    mlp = jnp.dot(h2, w3_ref[...], preferred_element_type=jnp.float32) \
        + b3_ref[...]

    y_ref[...] = jax.nn.sigmoid((fm + mlp)[:, 0])


def _tc_dense(u2, i2, d2, eu, ei, ed, year, goh, genre_table, fc_w, bias2,
              w1, b1_2, w2, b2_2, w3, b3_2, blk):
    grid = (B // blk,)
    blkspec = lambda shape: pl.BlockSpec(shape, lambda i: (i, 0))
    full = lambda shape: pl.BlockSpec(shape, lambda i: (0, 0))
    return pl.pallas_call(
        _tc_body,
        grid=grid,
        in_specs=[
            blkspec((blk, 1)),
            blkspec((blk, 1)),
            blkspec((blk, 1)),
            blkspec((blk, 128)),
            blkspec((blk, 128)),
            blkspec((blk, 128)),
            blkspec((blk, 1)),
            full((1, 32)),
            full((32, D)),
            full((4 * D, 1)),
            full((1, 1)),
            full((4 * D + 1, 64)),
            full((1, 64)),
            full((64, 32)),
            full((1, 32)),
            full((32, 1)),
            full((1, 1)),
        ],
        out_specs=pl.BlockSpec((blk,), lambda i: (i,)),
        out_shape=jax.ShapeDtypeStruct((B,), jnp.float32),
    )(u2, i2, d2, eu, ei, ed, year, goh, genre_table, fc_w, bias2, w1, b1_2,
      w2, b2_2, w3, b3_2)


def kernel(user, item, genre, director, year, user_table, item_table,
           genre_table, director_table, fc_w, bias, w1, b1, w2, b2, w3, b3):
    user = user.astype(jnp.int32)
    item = item.astype(jnp.int32)
    director = director.astype(jnp.int32)

    ut2 = _regroup(user_table.T)
    eu = _sc_gather(user, ut2)
    it2 = _regroup(item_table.T)
    ei = _sc_gather(item, it2)
    dt2 = _regroup(director_table.T)
    ed = _sc_gather(director, dt2)

    goh = (genre.reshape(-1)[0] == jnp.arange(32, dtype=genre.dtype)
           ).astype(jnp.float32).reshape(1, 32)

    return _tc_dense(
        user.reshape(-1, 1), item.reshape(-1, 1), director.reshape(-1, 1),
        eu, ei, ed, year, goh, genre_table,
        fc_w, bias.reshape(1, 1), w1, b1.reshape(1, -1), w2,
        b2.reshape(1, -1), w3, b3.reshape(1, 1), blk=2048)


# speedup vs baseline: 2.7256x; 2.7256x over previous
import functools

import jax
import jax.numpy as jnp
from jax import lax
from jax.experimental import pallas as pl
from jax.experimental.pallas import tpu as pltpu
from jax.experimental.pallas import tpu_sc as plsc

B = 16384
D = 16
G = 128 // D


def _regroup_body(xt_ref, out_ref):
    x = xt_ref[...]
    nq = x.shape[1] // 1024
    ident = jnp.where(
        lax.broadcasted_iota(jnp.int32, (128, 128), 0)
        == lax.broadcasted_iota(jnp.int32, (128, 128), 1), 1.0, 0.0)
    for q in range(nq):
        s = jnp.concatenate(
            [x[:, q * 1024 + t * 128:q * 1024 + (t + 1) * 128]
             for t in range(8)], axis=0)
        out_ref[q * 128:(q + 1) * 128, :] = lax.dot_general(
            ident, s, (((1,), (1,)), ((), ())),
            preferred_element_type=jnp.float32)


def _regroup(table_t, blk_i=131072):
    d, v = table_t.shape
    grid = (pl.cdiv(v, blk_i),)
    return pl.pallas_call(
        _regroup_body,
        grid=grid,
        in_specs=[pl.BlockSpec((d, blk_i), lambda i: (0, i))],
        out_specs=pl.BlockSpec((blk_i // 8, 128), lambda i: (i, 0)),
        out_shape=jax.ShapeDtypeStruct(
            (pl.cdiv(v, blk_i) * (blk_i // 8), 128), jnp.float32),
    )(table_t)


def _make_sc_gather(hmode):
    info = plsc.get_sparse_core_info()
    nc, ns = info.num_cores, info.num_subcores
    nw = nc * ns
    bpw = B // nw

    mesh = plsc.VectorSubcoreMesh(core_axis_name="c", subcore_axis_name="s")

    @functools.partial(
        pl.kernel,
        mesh=mesh,
        out_type=jax.ShapeDtypeStruct((B, 128), jnp.float32),
        scratch_types=[
            pltpu.VMEM((bpw,), jnp.int32),
            pltpu.VMEM((bpw,), jnp.int32),
            pltpu.VMEM((128, 128), jnp.float32),
            pltpu.VMEM((128, 128), jnp.float32),
            pltpu.SemaphoreType.DMA,
            pltpu.SemaphoreType.DMA,
        ],
    )
    def sc_gather(
        idx_hbm, tab_hbm, out,
        idx_v, hi_v, rows_a, rows_b, sem_a, sem_b,
    ):
        wid = lax.axis_index("s") * nc + lax.axis_index("c")
        base = wid * bpw
        sl = pl.ds(base, bpw)

        pltpu.sync_copy(idx_hbm.at[sl], idx_v)

        for k in range(bpw // 16):
            ks = pl.ds(k * 16, 16)
            ix = idx_v[ks]
            if hmode:
                hi_v[ks] = jnp.bitwise_or(
                    lax.shift_left(lax.shift_right_logical(ix, 10), 7),
                    jnp.bitwise_and(ix, 127))
            else:
                hi_v[ks] = lax.shift_right_logical(ix, 3)

        nch = bpw // 128
        rows = [rows_a, rows_b]
        sems = [sem_a, sem_b]
        cps = [None, None]
        for t in range(nch):
            cps[t % 2] = pltpu.async_copy(
                tab_hbm.at[hi_v.at[pl.ds(t * 128, 128)]],
                rows[t % 2], sems[t % 2])
            if t > 0:
                cps[(t - 1) % 2].wait()
                pltpu.sync_copy(rows[(t - 1) % 2],
                                out.at[pl.ds(base + (t - 1) * 128, 128)])
        cps[(nch - 1) % 2].wait()
        pltpu.sync_copy(rows[(nch - 1) % 2],
                        out.at[pl.ds(base + (nch - 1) * 128, 128)])

    return sc_gather


_sc_gather = _make_sc_gather(True)
_sc_gather_grouped = _make_sc_gather(False)


def _extract(rows128, idx, hmode=True):
    if hmode:
        off = jnp.bitwise_and(lax.shift_right_logical(idx, 7), G - 1)
    else:
        off = jnp.bitwise_and(idx, G - 1)
    out = jnp.zeros((rows128.shape[0], D), jnp.float32)
    for j in range(G):
        out = out + jnp.where(off == j, rows128[:, j * D:(j + 1) * D], 0.0)
    return out


def _tc_body(u_ref, i_ref, d_ref, eu_ref, ei_ref, ed_ref, year_ref, goh_ref,
             gt_ref, fcw_ref, bias_ref, w1_ref, b1_ref, w2_ref, b2_ref,
             w3_ref, b3_ref, y_ref):
    eu = _extract(eu_ref[...], u_ref[...])
    ei = _extract(ei_ref[...], i_ref[...])
    ed = _extract(ed_ref[...], d_ref[...])
    g = jnp.dot(goh_ref[...], gt_ref[...],
                preferred_element_type=jnp.float32)
    yr = year_ref[...]

    v = eu + ei + ed + g
    s = jnp.sum(v, axis=1, keepdims=True)
    ssq = jnp.sum(v * v, axis=1, keepdims=True)
    second = 0.5 * (s * s - ssq)

    fcw = fcw_ref[...]
    fm = (
        jnp.dot(eu, fcw[0:D], preferred_element_type=jnp.float32)
        + jnp.dot(ei, fcw[D:2 * D], preferred_element_type=jnp.float32)
        + jnp.dot(ed, fcw[3 * D:4 * D], preferred_element_type=jnp.float32)
        + jnp.dot(g, fcw[2 * D:3 * D], preferred_element_type=jnp.float32)
        + bias_ref[...]
        + second
        + yr
    )

    w1 = w1_ref[...]
    p = (
        jnp.dot(eu, w1[0:D], preferred_element_type=jnp.float32)
        + jnp.dot(ei, w1[D:2 * D], preferred_element_type=jnp.float32)
        + jnp.dot(ed, w1[2 * D:3 * D], preferred_element_type=jnp.float32)
        + jnp.dot(g, w1[3 * D:4 * D], preferred_element_type=jnp.float32)
        + yr * w1[4 * D:4 * D + 1]
        + b1_ref[...]
    )
    h1 = jnp.maximum(p, 0.0)
    h2 = jnp.maximum(
        jnp.dot(h1, w2_ref[...], preferred_element_type=jnp.float32)
        + b2_ref[...], 0.0)
    mlp = jnp.dot(h2, w3_ref[...], preferred_element_type=jnp.float32) \
        + b3_ref[...]

    y_ref[...] = jax.nn.sigmoid((fm + mlp)[:, 0])


def _tc_dense(u2, i2, d2, eu, ei, ed, year, goh, genre_table, fc_w, bias2,
              w1, b1_2, w2, b2_2, w3, b3_2, blk):
    grid = (B // blk,)
    blkspec = lambda shape: pl.BlockSpec(shape, lambda i: (i, 0))
    full = lambda shape: pl.BlockSpec(shape, lambda i: (0, 0))
    return pl.pallas_call(
        _tc_body,
        grid=grid,
        in_specs=[
            blkspec((blk, 1)),
            blkspec((blk, 1)),
            blkspec((blk, 1)),
            blkspec((blk, 128)),
            blkspec((blk, 128)),
            blkspec((blk, 128)),
            blkspec((blk, 1)),
            full((1, 32)),
            full((32, D)),
            full((4 * D, 1)),
            full((1, 1)),
            full((4 * D + 1, 64)),
            full((1, 64)),
            full((64, 32)),
            full((1, 32)),
            full((32, 1)),
            full((1, 1)),
        ],
        out_specs=pl.BlockSpec((blk,), lambda i: (i,)),
        out_shape=jax.ShapeDtypeStruct((B,), jnp.float32),
    )(u2, i2, d2, eu, ei, ed, year, goh, genre_table, fc_w, bias2, w1, b1_2,
      w2, b2_2, w3, b3_2)


def kernel(user, item, genre, director, year, user_table, item_table,
           genre_table, director_table, fc_w, bias, w1, b1, w2, b2, w3, b3):
    user = user.astype(jnp.int32)
    item = item.astype(jnp.int32)
    director = director.astype(jnp.int32)

    ut2 = _regroup(user_table.T)
    eu = _sc_gather(user, ut2)
    it2 = _regroup(item_table.T)
    ei = _sc_gather(item, it2)
    dt2 = _regroup(director_table.T)
    ed = _sc_gather(director, dt2)

    goh = (genre.reshape(-1)[0] == jnp.arange(32, dtype=genre.dtype)
           ).astype(jnp.float32).reshape(1, 32)

    return _tc_dense(
        user.reshape(-1, 1), item.reshape(-1, 1), director.reshape(-1, 1),
        eu, ei, ed, year, goh, genre_table,
        fc_w, bias.reshape(1, 1), w1, b1.reshape(1, -1), w2,
        b2.reshape(1, -1), w3, b3.reshape(1, 1), blk=2048)


# speedup vs baseline: 2.7301x; 1.0016x over previous
import functools

import jax
import jax.numpy as jnp
from jax import lax
from jax.experimental import pallas as pl
from jax.experimental.pallas import tpu as pltpu
from jax.experimental.pallas import tpu_sc as plsc

B = 16384
D = 16
G = 128 // D


def _regroup_body(xt_ref, out_ref):
    x = xt_ref[...]
    nq = x.shape[1] // 1024
    ident = jnp.where(
        lax.broadcasted_iota(jnp.int32, (128, 128), 0)
        == lax.broadcasted_iota(jnp.int32, (128, 128), 1), 1.0, 0.0)
    for q in range(nq):
        s = jnp.concatenate(
            [x[:, q * 1024 + t * 128:q * 1024 + (t + 1) * 128]
             for t in range(8)], axis=0)
        out_ref[q * 128:(q + 1) * 128, :] = lax.dot_general(
            ident, s, (((1,), (1,)), ((), ())),
            preferred_element_type=jnp.float32)


def _regroup(table_t, blk_i=131072):
    d, v = table_t.shape
    grid = (pl.cdiv(v, blk_i),)
    return pl.pallas_call(
        _regroup_body,
        grid=grid,
        in_specs=[pl.BlockSpec((d, blk_i), lambda i: (0, i))],
        out_specs=pl.BlockSpec((blk_i // 8, 128), lambda i: (i, 0)),
        out_shape=jax.ShapeDtypeStruct(
            (pl.cdiv(v, blk_i) * (blk_i // 8), 128), jnp.float32),
    )(table_t)


def _make_sc_gather(hmode):
    info = plsc.get_sparse_core_info()
    nc, ns = info.num_cores, info.num_subcores
    nw = nc * ns
    bpw = B // nw

    mesh = plsc.VectorSubcoreMesh(core_axis_name="c", subcore_axis_name="s")

    @functools.partial(
        pl.kernel,
        mesh=mesh,
        out_type=jax.ShapeDtypeStruct((B, 128), jnp.float32),
        scratch_types=[
            pltpu.VMEM((bpw,), jnp.int32),
            pltpu.VMEM((bpw,), jnp.int32),
            pltpu.VMEM((128, 128), jnp.float32),
            pltpu.VMEM((128, 128), jnp.float32),
            pltpu.SemaphoreType.DMA,
            pltpu.SemaphoreType.DMA,
        ],
    )
    def sc_gather(
        idx_hbm, tab_hbm, out,
        idx_v, hi_v, rows_a, rows_b, sem_a, sem_b,
    ):
        wid = lax.axis_index("s") * nc + lax.axis_index("c")
        base = wid * bpw
        sl = pl.ds(base, bpw)

        pltpu.sync_copy(idx_hbm.at[sl], idx_v)

        for k in range(bpw // 16):
            ks = pl.ds(k * 16, 16)
            ix = idx_v[ks]
            if hmode:
                hi_v[ks] = jnp.bitwise_or(
                    lax.shift_left(lax.shift_right_logical(ix, 10), 7),
                    jnp.bitwise_and(ix, 127))
            else:
                hi_v[ks] = lax.shift_right_logical(ix, 3)

        nch = bpw // 128
        rows = [rows_a, rows_b]
        sems = [sem_a, sem_b]
        cps = [None, None]
        for t in range(nch):
            cps[t % 2] = pltpu.async_copy(
                tab_hbm.at[hi_v.at[pl.ds(t * 128, 128)]],
                rows[t % 2], sems[t % 2])
            if t > 0:
                cps[(t - 1) % 2].wait()
                pltpu.sync_copy(rows[(t - 1) % 2],
                                out.at[pl.ds(base + (t - 1) * 128, 128)])
        cps[(nch - 1) % 2].wait()
        pltpu.sync_copy(rows[(nch - 1) % 2],
                        out.at[pl.ds(base + (nch - 1) * 128, 128)])

    return sc_gather


_sc_gather = _make_sc_gather(True)


def _extract(rows128, idx, hmode=True):
    if hmode:
        off = jnp.bitwise_and(lax.shift_right_logical(idx, 7), G - 1)
    else:
        off = jnp.bitwise_and(idx, G - 1)
    out = jnp.zeros((rows128.shape[0], D), jnp.float32)
    for j in range(G):
        out = out + jnp.where(off == j, rows128[:, j * D:(j + 1) * D], 0.0)
    return out


def _tc_body(u_ref, i_ref, d_ref, eu_ref, ei_ref, ed_ref, year_ref, goh_ref,
             gt_ref, fcw_ref, bias_ref, w1_ref, b1_ref, w2_ref, b2_ref,
             w3_ref, b3_ref, y_ref):
    eu = _extract(eu_ref[...], u_ref[...])
    ei = _extract(ei_ref[...], i_ref[...])
    ed = _extract(ed_ref[...], d_ref[...])
    g = jnp.dot(goh_ref[...], gt_ref[...],
                preferred_element_type=jnp.float32)
    yr = year_ref[...]

    v = eu + ei + ed + g
    s = jnp.sum(v, axis=1, keepdims=True)
    ssq = jnp.sum(v * v, axis=1, keepdims=True)
    second = 0.5 * (s * s - ssq)

    fcw = fcw_ref[...]
    fm = (
        jnp.dot(eu, fcw[0:D], preferred_element_type=jnp.float32)
        + jnp.dot(ei, fcw[D:2 * D], preferred_element_type=jnp.float32)
        + jnp.dot(ed, fcw[3 * D:4 * D], preferred_element_type=jnp.float32)
        + jnp.dot(g, fcw[2 * D:3 * D], preferred_element_type=jnp.float32)
        + bias_ref[...]
        + second
        + yr
    )

    w1 = w1_ref[...]
    p = (
        jnp.dot(eu, w1[0:D], preferred_element_type=jnp.float32)
        + jnp.dot(ei, w1[D:2 * D], preferred_element_type=jnp.float32)
        + jnp.dot(ed, w1[2 * D:3 * D], preferred_element_type=jnp.float32)
        + jnp.dot(g, w1[3 * D:4 * D], preferred_element_type=jnp.float32)
        + yr * w1[4 * D:4 * D + 1]
        + b1_ref[...]
    )
    h1 = jnp.maximum(p, 0.0)
    h2 = jnp.maximum(
        jnp.dot(h1, w2_ref[...], preferred_element_type=jnp.float32)
        + b2_ref[...], 0.0)
    mlp = jnp.dot(h2, w3_ref[...], preferred_element_type=jnp.float32) \
        + b3_ref[...]

    y_ref[...] = jax.nn.sigmoid((fm + mlp)[:, 0])


def _tc_dense(u2, i2, d2, eu, ei, ed, year, goh, genre_table, fc_w, bias2,
              w1, b1_2, w2, b2_2, w3, b3_2, blk):
    grid = (B // blk,)
    blkspec = lambda shape: pl.BlockSpec(shape, lambda i: (i, 0))
    full = lambda shape: pl.BlockSpec(shape, lambda i: (0, 0))
    return pl.pallas_call(
        _tc_body,
        grid=grid,
        in_specs=[
            blkspec((blk, 1)),
            blkspec((blk, 1)),
            blkspec((blk, 1)),
            blkspec((blk, 128)),
            blkspec((blk, 128)),
            blkspec((blk, 128)),
            blkspec((blk, 1)),
            full((1, 32)),
            full((32, D)),
            full((4 * D, 1)),
            full((1, 1)),
            full((4 * D + 1, 64)),
            full((1, 64)),
            full((64, 32)),
            full((1, 32)),
            full((32, 1)),
            full((1, 1)),
        ],
        out_specs=pl.BlockSpec((blk,), lambda i: (i,)),
        out_shape=jax.ShapeDtypeStruct((B,), jnp.float32),
    )(u2, i2, d2, eu, ei, ed, year, goh, genre_table, fc_w, bias2, w1, b1_2,
      w2, b2_2, w3, b3_2)


def kernel(user, item, genre, director, year, user_table, item_table,
           genre_table, director_table, fc_w, bias, w1, b1, w2, b2, w3, b3):
    user = user.astype(jnp.int32)
    item = item.astype(jnp.int32)
    director = director.astype(jnp.int32)

    ut2 = _regroup(user_table.T)
    eu = _sc_gather(user, ut2)
    it2 = _regroup(item_table.T)
    ei = _sc_gather(item, it2)
    dt2 = _regroup(director_table.T)
    ed = _sc_gather(director, dt2)

    goh = (genre.reshape(-1)[0] == jnp.arange(32, dtype=genre.dtype)
           ).astype(jnp.float32).reshape(1, 32)

    return _tc_dense(
        user.reshape(-1, 1), item.reshape(-1, 1), director.reshape(-1, 1),
        eu, ei, ed, year, goh, genre_table,
        fc_w, bias.reshape(1, 1), w1, b1.reshape(1, -1), w2,
        b2.reshape(1, -1), w3, b3.reshape(1, 1), blk=2048)
